# Initial kernel scaffold; baseline (speedup 1.0000x reference)
import functools

import jax
import jax.numpy as jnp
from jax import lax
from jax.experimental import pallas as pl
from jax.experimental.pallas import tpu as pltpu
from jax.experimental.pallas import tpu_sc as plsc

B, L, H, W = 2, 8, 128, 128
CI, CF = 32, 2
CB = CI + CF
CC = 48
NPX = H * W
R = B * L * NPX
NC, NS, LN = 2, 16, 16
NW = NC * NS
P = 128
GROUPS = P // LN
CPS = NPX // P


def _floorf(x):
    i = x.astype(jnp.int32)
    f = i.astype(jnp.float32)
    return jnp.where(f > x, f - 1.0, f)


def _make_round(s):
    nsl = B * (L - s)
    per_worker = nsl * CPS // NW

    mesh = plsc.VectorSubcoreMesh(
        core_axis_name="c", subcore_axis_name="s",
        num_cores=NC, num_subcores=NS)

    @functools.partial(
        pl.kernel,
        out_type=jax.ShapeDtypeStruct((R, CC), jnp.float32),
        mesh=mesh,
        scratch_types=[
            pltpu.VMEM((P, CC), jnp.float32),
            pltpu.VMEM((P, CC), jnp.float32),
            pltpu.VMEM((P, CC), jnp.float32),
            pltpu.VMEM((P, CC), jnp.float32),
            pltpu.VMEM((P, CC), jnp.float32),
            pltpu.VMEM((P, CC), jnp.float32),
            pltpu.VMEM((P,), jnp.int32),
            pltpu.VMEM((P,), jnp.int32),
            pltpu.VMEM((P,), jnp.int32),
            pltpu.VMEM((P,), jnp.int32),
            pltpu.VMEM((P,), jnp.float32),
            pltpu.VMEM((P,), jnp.float32),
            pltpu.VMEM((P,), jnp.float32),
            pltpu.VMEM((P,), jnp.float32),
            pltpu.SemaphoreType.DMA,
            pltpu.SemaphoreType.DMA,
            pltpu.SemaphoreType.DMA,
            pltpu.SemaphoreType.DMA,
        ],
    )
    def round_kernel(state_in, state_out, cur, t0, t1, t2, t3, orows,
                     i0, i1, i2, i3, w0, w1, w2, w3, s0, s1, s2, s3):
        wid = lax.axis_index("s") * NC + lax.axis_index("c")

        for b in range(B):
            for l in range(s):
                base = (b * L + l) * NPX

                @pl.loop(0, (NPX // NW) // P)
                def _cp(k2, base=base):
                    r0 = base + wid * (NPX // NW) + k2 * P
                    pltpu.sync_copy(state_in.at[pl.ds(r0, P)], cur)
                    pltpu.sync_copy(cur, state_out.at[pl.ds(r0, P)])

        @pl.loop(0, per_worker)
        def _chunk(k):
            g = k * NW + wid
            sl = g >> 7
            p0 = (g - sl * CPS) * P
            bb = (sl >= (L - s)).astype(jnp.int32)
            ll = sl - bb * (L - s) + s
            slice_cur = bb * L + ll
            row_cur = slice_cur * NPX + p0
            prev_base = (slice_cur - s) * NPX

            pltpu.sync_copy(state_in.at[pl.ds(row_cur, P)], cur)

            @pl.loop(0, GROUPS)
            def _idx(j):
                lane = lax.iota(jnp.int32, LN)
                loc = j * LN + lane
                pix = p0 + loc
                wi = pix & (W - 1)
                hi = pix >> 7
                fx = plsc.load_gather(cur, [loc, jnp.full((LN,), CI, jnp.int32)])
                fy = plsc.load_gather(cur, [loc, jnp.full((LN,), CI + 1, jnp.int32)])
                gx = wi.astype(jnp.float32) * (2.0 / W) + (1.0 / W - 1.0)
                gy = hi.astype(jnp.float32) * (2.0 / H) + (1.0 / H - 1.0)
                tx = gx + fx + 1.0
                tx = tx - 2.0 * _floorf(tx * 0.5)
                rx = tx * (W * 0.5) - 0.5
                ry = (gy + fy + 1.0) * (H * 0.5) - 0.5
                x0 = _floorf(rx)
                y0 = _floorf(ry)
                wx1 = rx - x0
                wx0 = 1.0 - wx1
                wy1 = ry - y0
                wy0 = 1.0 - wy1
                ix0 = x0.astype(jnp.int32)
                iy0 = y0.astype(jnp.int32)
                ix1 = ix0 + 1
                iy1 = iy0 + 1

                def tap(iy, ix, wgt, ib, wb):
                    valid = (ix >= 0) & (ix < W) & (iy >= 0) & (iy < H)
                    idx = (prev_base
                           + jnp.clip(iy, 0, H - 1) * W
                           + jnp.clip(ix, 0, W - 1))
                    ib[pl.ds(j * LN, LN)] = idx
                    wb[pl.ds(j * LN, LN)] = jnp.where(valid, wgt, 0.0)

                tap(iy0, ix0, wx0 * wy0, i0, w0)
                tap(iy0, ix1, wx1 * wy0, i1, w1)
                tap(iy1, ix0, wx0 * wy1, i2, w2)
                tap(iy1, ix1, wx1 * wy1, i3, w3)

            h0 = pltpu.async_copy(state_in.at[i0], t0, s0)
            h1 = pltpu.async_copy(state_in.at[i1], t1, s1)
            h2 = pltpu.async_copy(state_in.at[i2], t2, s2)
            h3 = pltpu.async_copy(state_in.at[i3], t3, s3)
            h0.wait()
            h1.wait()
            h2.wait()
            h3.wait()

            @pl.loop(0, GROUPS)
            def _blend(j):
                lane = lax.iota(jnp.int32, LN)
                rows = j * LN + lane
                wv0 = w0[pl.ds(j * LN, LN)]
                wv1 = w1[pl.ds(j * LN, LN)]
                wv2 = w2[pl.ds(j * LN, LN)]
                wv3 = w3[pl.ds(j * LN, LN)]
                for c in range(CB):
                    cc = jnp.full((LN,), c, jnp.int32)
                    acc = plsc.load_gather(cur, [rows, cc])
                    acc = acc + wv0 * plsc.load_gather(t0, [rows, cc])
                    acc = acc + wv1 * plsc.load_gather(t1, [rows, cc])
                    acc = acc + wv2 * plsc.load_gather(t2, [rows, cc])
                    acc = acc + wv3 * plsc.load_gather(t3, [rows, cc])
                    plsc.store_scatter(orows, [rows, cc], acc)

            pltpu.sync_copy(orows, state_out.at[pl.ds(row_cur, P)])

    return round_kernel


_ROUNDS = {s: _make_round(s) for s in (1, 2, 4)}


def kernel(flows, images):
    fl = flows.astype(jnp.float32)
    im = images.astype(jnp.float32)
    imgs_px = jnp.transpose(im.reshape(B, L, CI, NPX), (0, 1, 3, 2))
    flows_px = jnp.transpose(fl.reshape(B, L, CF, NPX), (0, 1, 3, 2))
    pad = jnp.zeros((B, L, NPX, CC - CB), jnp.float32)
    state = jnp.concatenate([imgs_px, flows_px, pad], axis=-1).reshape(R, CC)
    for s in (1, 2, 4):
        state = _ROUNDS[s](state)
    out = state.reshape(B, L, NPX, CC)[..., :CI]
    return jnp.transpose(out, (0, 1, 3, 2)).reshape(B, L, CI, H, W)


# speedup vs baseline: 1.4173x; 1.4173x over previous
import functools

import jax
import jax.numpy as jnp
from jax import lax
from jax.experimental import pallas as pl
from jax.experimental.pallas import tpu as pltpu
from jax.experimental.pallas import tpu_sc as plsc

B, L, H, W = 2, 8, 128, 128
CI, CF = 32, 2
CB = CI + CF
CC = 48
NPX = H * W
R = B * L * NPX
NC, NS, LN = 2, 16, 16
NW = NC * NS
P = 128
GROUPS = P // LN
CPS = NPX // P


def _floorf(x):
    i = x.astype(jnp.int32)
    f = i.astype(jnp.float32)
    return jnp.where(f > x, f - 1.0, f)


def _make_round(s):
    nsl = B * (L - s)
    per_worker = nsl * CPS // NW

    mesh = plsc.VectorSubcoreMesh(
        core_axis_name="c", subcore_axis_name="s",
        num_cores=NC, num_subcores=NS)

    @functools.partial(
        pl.kernel,
        out_type=jax.ShapeDtypeStruct((R, CC), jnp.float32),
        mesh=mesh,
        scratch_types=[
            pltpu.VMEM((P, CC), jnp.float32),
            pltpu.VMEM((P, CC), jnp.float32),
            pltpu.VMEM((P, CC), jnp.float32),
            pltpu.VMEM((P, CC), jnp.float32),
            pltpu.VMEM((P, CC), jnp.float32),
            pltpu.VMEM((P, CC), jnp.float32),
            pltpu.VMEM((P,), jnp.int32),
            pltpu.VMEM((P,), jnp.int32),
            pltpu.VMEM((P,), jnp.int32),
            pltpu.VMEM((P,), jnp.int32),
            pltpu.VMEM((P,), jnp.float32),
            pltpu.VMEM((P,), jnp.float32),
            pltpu.VMEM((P,), jnp.float32),
            pltpu.VMEM((P,), jnp.float32),
            pltpu.SemaphoreType.DMA,
            pltpu.SemaphoreType.DMA,
            pltpu.SemaphoreType.DMA,
            pltpu.SemaphoreType.DMA,
        ],
        compiler_params=pltpu.CompilerParams(
            needs_layout_passes=False, use_tc_tiling_on_sc=False),
    )
    def round_kernel(state_in, state_out, cur, t0, t1, t2, t3, orows,
                     i0, i1, i2, i3, w0, w1, w2, w3, s0, s1, s2, s3):
        wid = lax.axis_index("s") * NC + lax.axis_index("c")

        for b in range(B):
            for l in range(s):
                base = (b * L + l) * NPX

                @pl.loop(0, (NPX // NW) // P)
                def _cp(k2, base=base):
                    r0 = base + wid * (NPX // NW) + k2 * P
                    pltpu.sync_copy(state_in.at[pl.ds(r0, P)], cur)
                    pltpu.sync_copy(cur, state_out.at[pl.ds(r0, P)])

        @pl.loop(0, per_worker)
        def _chunk(k):
            g = k * NW + wid
            sl = g >> 7
            p0 = (g - sl * CPS) * P
            bb = (sl >= (L - s)).astype(jnp.int32)
            ll = sl - bb * (L - s) + s
            slice_cur = bb * L + ll
            row_cur = slice_cur * NPX + p0
            prev_base = (slice_cur - s) * NPX

            pltpu.sync_copy(state_in.at[pl.ds(row_cur, P)], cur)

            @pl.loop(0, GROUPS)
            def _idx(j):
                lane = lax.iota(jnp.int32, LN)
                loc = j * LN + lane
                pix = p0 + loc
                wi = pix & (W - 1)
                hi = pix >> 7
                fx = plsc.load_gather(cur, [loc, jnp.full((LN,), CI, jnp.int32)])
                fy = plsc.load_gather(cur, [loc, jnp.full((LN,), CI + 1, jnp.int32)])
                gx = wi.astype(jnp.float32) * (2.0 / W) + (1.0 / W - 1.0)
                gy = hi.astype(jnp.float32) * (2.0 / H) + (1.0 / H - 1.0)
                tx = gx + fx + 1.0
                tx = tx - 2.0 * _floorf(tx * 0.5)
                rx = tx * (W * 0.5) - 0.5
                ry = (gy + fy + 1.0) * (H * 0.5) - 0.5
                x0 = _floorf(rx)
                y0 = _floorf(ry)
                wx1 = rx - x0
                wx0 = 1.0 - wx1
                wy1 = ry - y0
                wy0 = 1.0 - wy1
                ix0 = x0.astype(jnp.int32)
                iy0 = y0.astype(jnp.int32)
                ix1 = ix0 + 1
                iy1 = iy0 + 1

                def tap(iy, ix, wgt, ib, wb):
                    valid = (ix >= 0) & (ix < W) & (iy >= 0) & (iy < H)
                    idx = (prev_base
                           + jnp.clip(iy, 0, H - 1) * W
                           + jnp.clip(ix, 0, W - 1))
                    ib[pl.ds(j * LN, LN)] = idx
                    wb[pl.ds(j * LN, LN)] = jnp.where(valid, wgt, 0.0)

                tap(iy0, ix0, wx0 * wy0, i0, w0)
                tap(iy0, ix1, wx1 * wy0, i1, w1)
                tap(iy1, ix0, wx0 * wy1, i2, w2)
                tap(iy1, ix1, wx1 * wy1, i3, w3)

            h0 = pltpu.async_copy(state_in.at[i0], t0, s0)
            h1 = pltpu.async_copy(state_in.at[i1], t1, s1)
            h2 = pltpu.async_copy(state_in.at[i2], t2, s2)
            h3 = pltpu.async_copy(state_in.at[i3], t3, s3)
            h0.wait()
            h1.wait()
            h2.wait()
            h3.wait()

            @pl.loop(0, GROUPS)
            def _blend(j):
                lane = lax.iota(jnp.int32, LN)
                rows = j * LN + lane
                wv0 = w0[pl.ds(j * LN, LN)]
                wv1 = w1[pl.ds(j * LN, LN)]
                wv2 = w2[pl.ds(j * LN, LN)]
                wv3 = w3[pl.ds(j * LN, LN)]
                for c in range(CB):
                    cc = jnp.full((LN,), c, jnp.int32)
                    acc = plsc.load_gather(cur, [rows, cc])
                    acc = acc + wv0 * plsc.load_gather(t0, [rows, cc])
                    acc = acc + wv1 * plsc.load_gather(t1, [rows, cc])
                    acc = acc + wv2 * plsc.load_gather(t2, [rows, cc])
                    acc = acc + wv3 * plsc.load_gather(t3, [rows, cc])
                    plsc.store_scatter(orows, [rows, cc], acc)

            pltpu.sync_copy(orows, state_out.at[pl.ds(row_cur, P)])

    return round_kernel


_ROUNDS = {s: _make_round(s) for s in (1, 2, 4)}


def kernel(flows, images):
    fl = flows.astype(jnp.float32)
    im = images.astype(jnp.float32)
    imgs_px = jnp.transpose(im.reshape(B, L, CI, NPX), (0, 1, 3, 2))
    flows_px = jnp.transpose(fl.reshape(B, L, CF, NPX), (0, 1, 3, 2))
    pad = jnp.zeros((B, L, NPX, CC - CB), jnp.float32)
    state = jnp.concatenate([imgs_px, flows_px, pad], axis=-1).reshape(R, CC)
    for s in (1, 2, 4):
        state = _ROUNDS[s](state)
    out = state.reshape(B, L, NPX, CC)[..., :CI]
    return jnp.transpose(out, (0, 1, 3, 2)).reshape(B, L, CI, H, W)


# speedup vs baseline: 1.7984x; 1.2690x over previous
import functools

import jax
import jax.numpy as jnp
from jax import lax
from jax.experimental import pallas as pl
from jax.experimental.pallas import tpu as pltpu
from jax.experimental.pallas import tpu_sc as plsc

B, L, H, W = 2, 8, 128, 128
CI, CF = 32, 2
CB = CI + CF
CC = 48
NPX = H * W
R = B * L * NPX
NC, NS, LN = 2, 16, 16
NW = NC * NS
P = 128
GROUPS = P // LN
CPS = NPX // P


def _floorf(x):
    i = x.astype(jnp.int32)
    f = i.astype(jnp.float32)
    return jnp.where(f > x, f - 1.0, f)


def _make_round(s):
    nsl = B * (L - s)
    pw = nsl * CPS // NW

    mesh = plsc.VectorSubcoreMesh(
        core_axis_name="c", subcore_axis_name="s",
        num_cores=NC, num_subcores=NS)

    scratch = (
        [pltpu.VMEM((P, CC), jnp.float32)] * 3
        + [pltpu.VMEM((P, CC), jnp.float32)] * 3
        + [pltpu.VMEM((P, CC), jnp.float32)] * 12
        + [pltpu.VMEM((P,), jnp.int32)] * 12
        + [pltpu.VMEM((P,), jnp.float32)] * 12
        + [pltpu.SemaphoreType.DMA] * 18
    )

    @functools.partial(
        pl.kernel,
        out_type=jax.ShapeDtypeStruct((R, CC), jnp.float32),
        mesh=mesh,
        scratch_types=scratch,
        compiler_params=pltpu.CompilerParams(
            needs_layout_passes=False, use_tc_tiling_on_sc=False),
    )
    def round_kernel(state_in, state_out, *scr):
        cur = scr[0:3]
        oro = scr[3:6]
        taps = [scr[6 + 4 * q:10 + 4 * q] for q in range(3)]
        ib = [scr[18 + 4 * q:22 + 4 * q] for q in range(3)]
        wb = [scr[30 + 4 * q:34 + 4 * q] for q in range(3)]
        semc = scr[42:45]
        semo = scr[45:48]
        semg = [scr[48 + 4 * q:52 + 4 * q] for q in range(3)]

        wid = lax.axis_index("s") * NC + lax.axis_index("c")

        for b in range(B):
            for l in range(s):
                base = (b * L + l) * NPX + wid * (NPX // NW)
                for t in range(4):
                    pltpu.async_copy(state_in.at[pl.ds(base + t * P, P)],
                                     taps[0][t], semg[0][t])
                for t in range(4):
                    pltpu.make_async_copy(state_in.at[pl.ds(0, P)],
                                          taps[0][t], semg[0][t]).wait()
                    pltpu.async_copy(taps[0][t],
                                     state_out.at[pl.ds(base + t * P, P)],
                                     semg[0][t])
                for t in range(4):
                    pltpu.make_async_copy(state_in.at[pl.ds(0, P)],
                                          taps[0][t], semg[0][t]).wait()

        def coords(k):
            g = k * NW + wid
            sl = g >> 7
            p0 = (g - sl * CPS) * P
            bb = (sl >= (L - s)).astype(jnp.int32)
            ll = sl - bb * (L - s) + s
            slice_cur = bb * L + ll
            return slice_cur * NPX + p0, (slice_cur - s) * NPX, p0

        def fire_cur(k, q):
            rc, _, _ = coords(k)
            pltpu.async_copy(state_in.at[pl.ds(rc, P)], cur[q], semc[q])
            pltpu.async_copy(state_in.at[pl.ds(rc, P)], oro[q], semo[q])

        def idx_pass(k, q):
            _, prev_base, p0 = coords(k)

            @pl.loop(0, GROUPS)
            def _idx(j):
                lane = lax.iota(jnp.int32, LN)
                loc = j * LN + lane
                pix = p0 + loc
                wi = pix & (W - 1)
                hi = pix >> 7
                fx = plsc.load_gather(cur[q], [loc, jnp.full((LN,), CI, jnp.int32)])
                fy = plsc.load_gather(cur[q], [loc, jnp.full((LN,), CI + 1, jnp.int32)])
                gx = wi.astype(jnp.float32) * (2.0 / W) + (1.0 / W - 1.0)
                gy = hi.astype(jnp.float32) * (2.0 / H) + (1.0 / H - 1.0)
                tx = gx + fx + 1.0
                tx = tx - 2.0 * _floorf(tx * 0.5)
                rx = tx * (W * 0.5) - 0.5
                ry = (gy + fy + 1.0) * (H * 0.5) - 0.5
                x0 = _floorf(rx)
                y0 = _floorf(ry)
                wx1 = rx - x0
                wx0 = 1.0 - wx1
                wy1 = ry - y0
                wy0 = 1.0 - wy1
                ix0 = x0.astype(jnp.int32)
                iy0 = y0.astype(jnp.int32)
                ix1 = ix0 + 1
                iy1 = iy0 + 1

                def tap(iy, ix, wgt, t):
                    valid = (ix >= 0) & (ix < W) & (iy >= 0) & (iy < H)
                    idx = (prev_base
                           + jnp.clip(iy, 0, H - 1) * W
                           + jnp.clip(ix, 0, W - 1))
                    ib[q][t][pl.ds(j * LN, LN)] = idx
                    wb[q][t][pl.ds(j * LN, LN)] = jnp.where(valid, wgt, 0.0)

                tap(iy0, ix0, wx0 * wy0, 0)
                tap(iy0, ix1, wx1 * wy0, 1)
                tap(iy1, ix0, wx0 * wy1, 2)
                tap(iy1, ix1, wx1 * wy1, 3)

        def blend(k, q):
            rc, _, _ = coords(k)
            for t in range(4):
                pltpu.make_async_copy(state_in.at[pl.ds(0, P)],
                                      taps[q][t], semg[q][t]).wait()
            pltpu.make_async_copy(state_in.at[pl.ds(0, P)],
                                  oro[q], semo[q]).wait()

            @pl.loop(0, GROUPS)
            def _blend(j):
                lane = lax.iota(jnp.int32, LN)
                rows = j * LN + lane
                wv0 = wb[q][0][pl.ds(j * LN, LN)]
                wv1 = wb[q][1][pl.ds(j * LN, LN)]
                wv2 = wb[q][2][pl.ds(j * LN, LN)]
                wv3 = wb[q][3][pl.ds(j * LN, LN)]
                for c in range(CB):
                    cc = jnp.full((LN,), c, jnp.int32)
                    acc = wv0 * plsc.load_gather(taps[q][0], [rows, cc])
                    acc = acc + wv1 * plsc.load_gather(taps[q][1], [rows, cc])
                    acc = acc + wv2 * plsc.load_gather(taps[q][2], [rows, cc])
                    acc = acc + wv3 * plsc.load_gather(taps[q][3], [rows, cc])
                    plsc.addupdate_scatter(oro[q], [rows, cc], acc)

            pltpu.async_copy(oro[q], state_out.at[pl.ds(rc, P)], semo[q])

        @pl.when(pw > 0)
        def _():
            fire_cur(0, 0)

        nbody = (pw + 4) // 3

        @pl.loop(0, nbody)
        def _outer(kk):
            for u in range(3):
                k = kk * 3 + u
                p1 = (u + 1) % 3
                p2 = (u + 2) % 3

                @pl.when(k < pw)
                def _(k=k, u=u):
                    pltpu.make_async_copy(state_in.at[pl.ds(0, P)],
                                          cur[u], semc[u]).wait()
                    idx_pass(k, u)
                    _, _, _ = coords(k)
                    for t in range(4):
                        pltpu.async_copy(state_in.at[ib[u][t]],
                                         taps[u][t], semg[u][t])

                @pl.when((k >= 2) & (k <= pw + 1))
                def _(k=k, p1=p1):
                    pltpu.make_async_copy(state_in.at[pl.ds(0, P)],
                                          oro[p1], semo[p1]).wait()

                @pl.when(k <= pw - 2)
                def _(k=k, p1=p1):
                    fire_cur(k + 1, p1)

                @pl.when((k >= 1) & (k <= pw))
                def _(k=k, p2=p2):
                    blend(k - 1, p2)

    return round_kernel


_ROUNDS = {s: _make_round(s) for s in (1, 2, 4)}


def kernel(flows, images):
    fl = flows.astype(jnp.float32)
    im = images.astype(jnp.float32)
    imgs_px = jnp.transpose(im.reshape(B, L, CI, NPX), (0, 1, 3, 2))
    flows_px = jnp.transpose(fl.reshape(B, L, CF, NPX), (0, 1, 3, 2))
    pad = jnp.zeros((B, L, NPX, CC - CB), jnp.float32)
    state = jnp.concatenate([imgs_px, flows_px, pad], axis=-1).reshape(R, CC)
    for s in (1, 2, 4):
        state = _ROUNDS[s](state)
    out = state.reshape(B, L, NPX, CC)[..., :CI]
    return jnp.transpose(out, (0, 1, 3, 2)).reshape(B, L, CI, H, W)


# speedup vs baseline: 2.1236x; 1.1808x over previous
import functools

import jax
import jax.numpy as jnp
from jax import lax
from jax.experimental import pallas as pl
from jax.experimental.pallas import tpu as pltpu
from jax.experimental.pallas import tpu_sc as plsc

B, L, H, W = 2, 8, 128, 128
CI, CF = 32, 2
CB = CI + CF
CC = 48
NPX = H * W
R = B * L * NPX
NC, NS, LN = 2, 16, 16
NW = NC * NS
P = 128
GROUPS = P // LN
CPS = NPX // P


def _floorf(x):
    i = x.astype(jnp.int32)
    f = i.astype(jnp.float32)
    return jnp.where(f > x, f - 1.0, f)


def _make_round(s):
    nsl = B * (L - s)
    pw = nsl * CPS // NW

    mesh = plsc.VectorSubcoreMesh(
        core_axis_name="c", subcore_axis_name="s",
        num_cores=NC, num_subcores=NS)

    scratch = (
        [pltpu.VMEM((P, CC), jnp.float32)] * 3
        + [pltpu.VMEM((P, CC), jnp.float32)] * 3
        + [pltpu.VMEM((P, CC), jnp.float32)] * 12
        + [pltpu.VMEM((P,), jnp.int32)] * 12
        + [pltpu.VMEM((P,), jnp.float32)] * 12
        + [pltpu.SemaphoreType.DMA] * 18
    )

    @functools.partial(
        pl.kernel,
        out_type=jax.ShapeDtypeStruct((R, CC), jnp.float32),
        mesh=mesh,
        scratch_types=scratch,
        compiler_params=pltpu.CompilerParams(
            needs_layout_passes=False, use_tc_tiling_on_sc=False),
    )
    def round_kernel(state_in, state_out, *scr):
        cur = scr[0:3]
        oro = scr[3:6]
        taps = [scr[6 + 4 * q:10 + 4 * q] for q in range(3)]
        ib = [scr[18 + 4 * q:22 + 4 * q] for q in range(3)]
        wb = [scr[30 + 4 * q:34 + 4 * q] for q in range(3)]
        semc = scr[42:45]
        semo = scr[45:48]
        semg = [scr[48 + 4 * q:52 + 4 * q] for q in range(3)]

        wid = lax.axis_index("s") * NC + lax.axis_index("c")

        for b in range(B):
            for l in range(s):
                base = (b * L + l) * NPX + wid * (NPX // NW)
                for t in range(4):
                    pltpu.async_copy(state_in.at[pl.ds(base + t * P, P)],
                                     taps[0][t], semg[0][t])
                for t in range(4):
                    pltpu.make_async_copy(state_in.at[pl.ds(0, P)],
                                          taps[0][t], semg[0][t]).wait()
                    pltpu.async_copy(taps[0][t],
                                     state_out.at[pl.ds(base + t * P, P)],
                                     semg[0][t])
                for t in range(4):
                    pltpu.make_async_copy(state_in.at[pl.ds(0, P)],
                                          taps[0][t], semg[0][t]).wait()

        def coords(k):
            g = k * NW + wid
            sl = g >> 7
            p0 = (g - sl * CPS) * P
            bb = (sl >= (L - s)).astype(jnp.int32)
            ll = sl - bb * (L - s) + s
            slice_cur = bb * L + ll
            return slice_cur * NPX + p0, (slice_cur - s) * NPX, p0

        def fire_cur(k, q):
            rc, _, _ = coords(k)
            pltpu.async_copy(state_in.at[pl.ds(rc, P)], cur[q], semc[q])
            pltpu.async_copy(state_in.at[pl.ds(rc, P)], oro[q], semo[q])

        def idx_pass(k, q):
            _, prev_base, p0 = coords(k)

            @plsc.parallel_loop(0, GROUPS)
            def _idx(j):
                lane = lax.iota(jnp.int32, LN)
                loc = j * LN + lane
                pix = p0 + loc
                wi = pix & (W - 1)
                hi = pix >> 7
                fx = plsc.load_gather(cur[q], [loc, jnp.full((LN,), CI, jnp.int32)])
                fy = plsc.load_gather(cur[q], [loc, jnp.full((LN,), CI + 1, jnp.int32)])
                gx = wi.astype(jnp.float32) * (2.0 / W) + (1.0 / W - 1.0)
                gy = hi.astype(jnp.float32) * (2.0 / H) + (1.0 / H - 1.0)
                tx = gx + fx + 1.0
                tx = tx - 2.0 * _floorf(tx * 0.5)
                rx = tx * (W * 0.5) - 0.5
                ry = (gy + fy + 1.0) * (H * 0.5) - 0.5
                x0 = _floorf(rx)
                y0 = _floorf(ry)
                wx1 = rx - x0
                wx0 = 1.0 - wx1
                wy1 = ry - y0
                wy0 = 1.0 - wy1
                ix0 = x0.astype(jnp.int32)
                iy0 = y0.astype(jnp.int32)
                ix1 = ix0 + 1
                iy1 = iy0 + 1

                def tap(iy, ix, wgt, t):
                    valid = (ix >= 0) & (ix < W) & (iy >= 0) & (iy < H)
                    idx = (prev_base
                           + jnp.clip(iy, 0, H - 1) * W
                           + jnp.clip(ix, 0, W - 1))
                    ib[q][t][pl.ds(j * LN, LN)] = idx
                    wb[q][t][pl.ds(j * LN, LN)] = jnp.where(valid, wgt, 0.0)

                tap(iy0, ix0, wx0 * wy0, 0)
                tap(iy0, ix1, wx1 * wy0, 1)
                tap(iy1, ix0, wx0 * wy1, 2)
                tap(iy1, ix1, wx1 * wy1, 3)

        def blend(k, q):
            rc, _, _ = coords(k)
            for t in range(4):
                pltpu.make_async_copy(state_in.at[pl.ds(0, P)],
                                      taps[q][t], semg[q][t]).wait()
            pltpu.make_async_copy(state_in.at[pl.ds(0, P)],
                                  oro[q], semo[q]).wait()

            @plsc.parallel_loop(0, GROUPS)
            def _blend(j):
                lane = lax.iota(jnp.int32, LN)
                rows = j * LN + lane
                wv0 = wb[q][0][pl.ds(j * LN, LN)]
                wv1 = wb[q][1][pl.ds(j * LN, LN)]
                wv2 = wb[q][2][pl.ds(j * LN, LN)]
                wv3 = wb[q][3][pl.ds(j * LN, LN)]
                for c in range(CB):
                    cc = jnp.full((LN,), c, jnp.int32)
                    a01 = (wv0 * plsc.load_gather(taps[q][0], [rows, cc])
                           + wv1 * plsc.load_gather(taps[q][1], [rows, cc]))
                    a23 = (wv2 * plsc.load_gather(taps[q][2], [rows, cc])
                           + wv3 * plsc.load_gather(taps[q][3], [rows, cc]))
                    plsc.addupdate_scatter(oro[q], [rows, cc], a01 + a23)

            pltpu.async_copy(oro[q], state_out.at[pl.ds(rc, P)], semo[q])

        @pl.when(pw > 0)
        def _():
            fire_cur(0, 0)

        nbody = (pw + 4) // 3

        @pl.loop(0, nbody)
        def _outer(kk):
            for u in range(3):
                k = kk * 3 + u
                p1 = (u + 1) % 3
                p2 = (u + 2) % 3

                @pl.when(k < pw)
                def _(k=k, u=u):
                    pltpu.make_async_copy(state_in.at[pl.ds(0, P)],
                                          cur[u], semc[u]).wait()
                    idx_pass(k, u)
                    _, _, _ = coords(k)
                    for t in range(4):
                        pltpu.async_copy(state_in.at[ib[u][t]],
                                         taps[u][t], semg[u][t])

                @pl.when((k >= 2) & (k <= pw + 1))
                def _(k=k, p1=p1):
                    pltpu.make_async_copy(state_in.at[pl.ds(0, P)],
                                          oro[p1], semo[p1]).wait()

                @pl.when(k <= pw - 2)
                def _(k=k, p1=p1):
                    fire_cur(k + 1, p1)

                @pl.when((k >= 1) & (k <= pw))
                def _(k=k, p2=p2):
                    blend(k - 1, p2)

    return round_kernel


_ROUNDS = {s: _make_round(s) for s in (1, 2, 4)}


def kernel(flows, images):
    fl = flows.astype(jnp.float32)
    im = images.astype(jnp.float32)
    imgs_px = jnp.transpose(im.reshape(B, L, CI, NPX), (0, 1, 3, 2))
    flows_px = jnp.transpose(fl.reshape(B, L, CF, NPX), (0, 1, 3, 2))
    pad = jnp.zeros((B, L, NPX, CC - CB), jnp.float32)
    state = jnp.concatenate([imgs_px, flows_px, pad], axis=-1).reshape(R, CC)
    for s in (1, 2, 4):
        state = _ROUNDS[s](state)
    out = state.reshape(B, L, NPX, CC)[..., :CI]
    return jnp.transpose(out, (0, 1, 3, 2)).reshape(B, L, CI, H, W)


# speedup vs baseline: 2.1514x; 1.0131x over previous
import functools

import jax
import jax.numpy as jnp
from jax import lax
from jax.experimental import pallas as pl
from jax.experimental.pallas import tpu as pltpu
from jax.experimental.pallas import tpu_sc as plsc

B, L, H, W = 2, 8, 128, 128
CI, CF = 32, 2
CB = CI + CF
CC = 48
NPX = H * W
R = B * L * NPX
NC, NS, LN = 2, 16, 16
NW = NC * NS
P = 128
GROUPS = P // LN
CPS = NPX // P


def _floorf(x):
    i = x.astype(jnp.int32)
    f = i.astype(jnp.float32)
    return jnp.where(f > x, f - 1.0, f)


def _make_round(s):
    nsl = B * (L - s)
    pw = nsl * CPS // NW

    mesh = plsc.VectorSubcoreMesh(
        core_axis_name="c", subcore_axis_name="s",
        num_cores=NC, num_subcores=NS)

    scratch = (
        [pltpu.VMEM((P, CC), jnp.float32)] * 3
        + [pltpu.VMEM((P, CC), jnp.float32)] * 3
        + [pltpu.VMEM((P, CC), jnp.float32)] * 12
        + [pltpu.VMEM((P,), jnp.int32)] * 12
        + [pltpu.VMEM((P,), jnp.float32)] * 12
        + [pltpu.SemaphoreType.DMA] * 18
    )

    @functools.partial(
        pl.kernel,
        out_type=jax.ShapeDtypeStruct((R, CC), jnp.float32),
        mesh=mesh,
        scratch_types=scratch,
        compiler_params=pltpu.CompilerParams(
            needs_layout_passes=False, use_tc_tiling_on_sc=False),
    )
    def round_kernel(state_in, state_out, *scr):
        cur = scr[0:3]
        oro = scr[3:6]
        taps = [scr[6 + 4 * q:10 + 4 * q] for q in range(3)]
        ib = [scr[18 + 4 * q:22 + 4 * q] for q in range(3)]
        wb = [scr[30 + 4 * q:34 + 4 * q] for q in range(3)]
        semc = scr[42:45]
        semo = scr[45:48]
        semg = [scr[48 + 4 * q:52 + 4 * q] for q in range(3)]

        wid = lax.axis_index("s") * NC + lax.axis_index("c")

        for b in range(B):
            for l in range(s):
                base = (b * L + l) * NPX + wid * (NPX // NW)
                for t in range(4):
                    pltpu.async_copy(state_in.at[pl.ds(base + t * P, P)],
                                     taps[0][t], semg[0][t])
                for t in range(4):
                    pltpu.make_async_copy(state_in.at[pl.ds(0, P)],
                                          taps[0][t], semg[0][t]).wait()
                    pltpu.async_copy(taps[0][t],
                                     state_out.at[pl.ds(base + t * P, P)],
                                     semg[0][t])
                for t in range(4):
                    pltpu.make_async_copy(state_in.at[pl.ds(0, P)],
                                          taps[0][t], semg[0][t]).wait()

        def coords(k):
            g = k * NW + wid
            sl = g >> 7
            p0 = (g - sl * CPS) * P
            bb = (sl >= (L - s)).astype(jnp.int32)
            ll = sl - bb * (L - s) + s
            slice_cur = bb * L + ll
            return slice_cur * NPX + p0, (slice_cur - s) * NPX, p0

        def fire_cur(k, q):
            rc, _, _ = coords(k)
            pltpu.async_copy(state_in.at[pl.ds(rc, P)], cur[q], semc[q])
            pltpu.async_copy(state_in.at[pl.ds(rc, P)], oro[q], semo[q])

        def idx_pass(k, q):
            _, prev_base, p0 = coords(k)

            @plsc.parallel_loop(0, GROUPS)
            def _idx(j):
                lane = lax.iota(jnp.int32, LN)
                loc = j * LN + lane
                pix = p0 + loc
                wi = pix & (W - 1)
                hi = pix >> 7
                fx = plsc.load_gather(cur[q], [loc, jnp.full((LN,), CI, jnp.int32)])
                fy = plsc.load_gather(cur[q], [loc, jnp.full((LN,), CI + 1, jnp.int32)])
                gx = wi.astype(jnp.float32) * (2.0 / W) + (1.0 / W - 1.0)
                gy = hi.astype(jnp.float32) * (2.0 / H) + (1.0 / H - 1.0)
                tx = gx + fx + 1.0
                tx = tx - 2.0 * _floorf(tx * 0.5)
                rx = tx * (W * 0.5) - 0.5
                ry = (gy + fy + 1.0) * (H * 0.5) - 0.5
                x0 = _floorf(rx)
                y0 = _floorf(ry)
                wx1 = rx - x0
                wx0 = 1.0 - wx1
                wy1 = ry - y0
                wy0 = 1.0 - wy1
                ix0 = x0.astype(jnp.int32)
                iy0 = y0.astype(jnp.int32)
                ix1 = ix0 + 1
                iy1 = iy0 + 1

                def tap(iy, ix, wgt, t):
                    valid = (ix >= 0) & (ix < W) & (iy >= 0) & (iy < H)
                    idx = (prev_base
                           + jnp.clip(iy, 0, H - 1) * W
                           + jnp.clip(ix, 0, W - 1))
                    ib[q][t][pl.ds(j * LN, LN)] = idx
                    wb[q][t][pl.ds(j * LN, LN)] = jnp.where(valid, wgt, 0.0)

                tap(iy0, ix0, wx0 * wy0, 0)
                tap(iy0, ix1, wx1 * wy0, 1)
                tap(iy1, ix0, wx0 * wy1, 2)
                tap(iy1, ix1, wx1 * wy1, 3)

        def blend(k, q):
            rc, _, _ = coords(k)
            for t in range(4):
                pltpu.make_async_copy(state_in.at[pl.ds(0, P)],
                                      taps[q][t], semg[q][t]).wait()
            pltpu.make_async_copy(state_in.at[pl.ds(0, P)],
                                  oro[q], semo[q]).wait()

            @plsc.parallel_loop(0, GROUPS, unroll=2)
            def _blend(j):
                lane = lax.iota(jnp.int32, LN)
                rows = j * LN + lane
                wv0 = wb[q][0][pl.ds(j * LN, LN)]
                wv1 = wb[q][1][pl.ds(j * LN, LN)]
                wv2 = wb[q][2][pl.ds(j * LN, LN)]
                wv3 = wb[q][3][pl.ds(j * LN, LN)]
                for c in range(CB):
                    cc = jnp.full((LN,), c, jnp.int32)
                    a01 = (wv0 * plsc.load_gather(taps[q][0], [rows, cc])
                           + wv1 * plsc.load_gather(taps[q][1], [rows, cc]))
                    a23 = (wv2 * plsc.load_gather(taps[q][2], [rows, cc])
                           + wv3 * plsc.load_gather(taps[q][3], [rows, cc]))
                    plsc.addupdate_scatter(oro[q], [rows, cc], a01 + a23)

            pltpu.async_copy(oro[q], state_out.at[pl.ds(rc, P)], semo[q])

        @pl.when(pw > 0)
        def _():
            fire_cur(0, 0)

        nbody = (pw + 4) // 3

        @pl.loop(0, nbody)
        def _outer(kk):
            for u in range(3):
                k = kk * 3 + u
                p1 = (u + 1) % 3
                p2 = (u + 2) % 3

                @pl.when(k < pw)
                def _(k=k, u=u):
                    pltpu.make_async_copy(state_in.at[pl.ds(0, P)],
                                          cur[u], semc[u]).wait()
                    idx_pass(k, u)
                    _, _, _ = coords(k)
                    for t in range(4):
                        pltpu.async_copy(state_in.at[ib[u][t]],
                                         taps[u][t], semg[u][t])

                @pl.when((k >= 2) & (k <= pw + 1))
                def _(k=k, p1=p1):
                    pltpu.make_async_copy(state_in.at[pl.ds(0, P)],
                                          oro[p1], semo[p1]).wait()

                @pl.when(k <= pw - 2)
                def _(k=k, p1=p1):
                    fire_cur(k + 1, p1)

                @pl.when((k >= 1) & (k <= pw))
                def _(k=k, p2=p2):
                    blend(k - 1, p2)

    return round_kernel


_ROUNDS = {s: _make_round(s) for s in (1, 2, 4)}


def kernel(flows, images):
    fl = flows.astype(jnp.float32)
    im = images.astype(jnp.float32)
    imgs_px = jnp.transpose(im.reshape(B, L, CI, NPX), (0, 1, 3, 2))
    flows_px = jnp.transpose(fl.reshape(B, L, CF, NPX), (0, 1, 3, 2))
    pad = jnp.zeros((B, L, NPX, CC - CB), jnp.float32)
    state = jnp.concatenate([imgs_px, flows_px, pad], axis=-1).reshape(R, CC)
    for s in (1, 2, 4):
        state = _ROUNDS[s](state)
    out = state.reshape(B, L, NPX, CC)[..., :CI]
    return jnp.transpose(out, (0, 1, 3, 2)).reshape(B, L, CI, H, W)
